# baseline (speedup 1.0000x reference)
import functools

import jax
import jax.numpy as jnp
from jax import lax
from jax.experimental import pallas as pl
from jax.experimental.pallas import tpu as pltpu
from jax.experimental.pallas import tpu_sc as plsc

B, S = 4096, 200
D = 64
NW = 32
BPW = B // NW
DT, DS, BL = D // 8, 8, 128


def _make_kernel():
    mesh = plsc.VectorSubcoreMesh(core_axis_name="c", subcore_axis_name="s")

    @functools.partial(
        pl.kernel,
        out_type=jax.ShapeDtypeStruct((S, DT, NW, DS, BL), jnp.float32),
        mesh=mesh,
        scratch_types=[
            pltpu.VMEM((BPW, S), jnp.int32),
            pltpu.VMEM((S, BPW), jnp.int32),
            pltpu.VMEM((2, BPW, D), jnp.float32),
            pltpu.VMEM((2, DT, DS, BL), jnp.float32),
            pltpu.SemaphoreType.DMA((2,)),
            pltpu.SemaphoreType.DMA((2,)),
        ],
        compiler_params=pltpu.CompilerParams(
            use_tc_tiling_on_sc=False, needs_layout_passes=False
        ),
    )
    def emb(tid_hbm, table_hbm, out_hbm, idx_v, idx_t, rows_v, xt_v, gsem, wsem):
        wid = lax.axis_index("s") * 2 + lax.axis_index("c")
        pltpu.sync_copy(tid_hbm.at[pl.ds(wid * BPW, BPW)], idx_v)

        lanes = lax.iota(jnp.int32, 16)
        row_idx = [lanes + j * 16 for j in range(BPW // 16)]
        zeros16 = jnp.full((16,), 0, jnp.int32)

        @plsc.parallel_loop(0, S, unroll=4)
        def _(s):
            col = zeros16 + s
            for j in range(BPW // 16):
                v = plsc.load_gather(idx_v, [row_idx[j], col])
                idx_t[s, pl.ds(j * 16, 16)] = v

        def transpose_rows(slot):
            @plsc.parallel_loop(0, D, unroll=2)
            def _(d):
                col = zeros16 + d
                dt = d // DS
                ds = d % DS
                vs = [
                    plsc.load_gather(rows_v.at[slot], [row_idx[j], col])
                    for j in range(BPW // 16)
                ]
                for j in range(BPW // 16):
                    xt_v[slot, dt, ds, pl.ds(j * 16, 16)] = vs[j]

        def gather(s, slot):
            pltpu.async_copy(
                table_hbm.at[idx_t.at[s]], rows_v.at[slot], gsem.at[slot]
            )

        def gather_wait(s, slot):
            pltpu.make_async_copy(
                table_hbm.at[idx_t.at[s]], rows_v.at[slot], gsem.at[slot]
            ).wait()

        def writeback(s, slot):
            pltpu.async_copy(
                xt_v.at[slot], out_hbm.at[s, :, wid], wsem.at[slot]
            )

        def writeback_wait(s, slot):
            pltpu.make_async_copy(
                xt_v.at[slot], out_hbm.at[s, :, wid], wsem.at[slot]
            ).wait()

        gather(0, 0)

        def body(i, _):
            s0 = i * 2
            for slot in (0, 1):
                s = s0 + slot
                nxt = s + 1

                @pl.when(nxt < S)
                def _():
                    gather(nxt, 1 - slot)

                gather_wait(s, slot)

                @pl.when(s >= 2)
                def _():
                    writeback_wait(s - 2, slot)

                transpose_rows(slot)
                writeback(s, slot)
            return ()

        lax.fori_loop(0, S // 2, body, ())
        writeback_wait(S - 2, 0)
        writeback_wait(S - 1, 1)

    return emb


_emb = _make_kernel()


@jax.jit
def kernel(token_ids, weight):
    out5 = _emb(token_ids, weight)
    return out5.transpose(2, 4, 0, 1, 3).reshape(B, S, D)


# speedup vs baseline: 1.2258x; 1.2258x over previous
import functools

import jax
import jax.numpy as jnp
from jax import lax
from jax.experimental import pallas as pl
from jax.experimental.pallas import tpu as pltpu
from jax.experimental.pallas import tpu_sc as plsc

B, S = 4096, 200
D = 64
NW = 32
SEQ_PER_W = B // NW
G = 8
NGRP = SEQ_PER_W // G


def _make_kernel():
    mesh = plsc.VectorSubcoreMesh(core_axis_name="c", subcore_axis_name="s")

    @functools.partial(
        pl.kernel,
        out_type=jax.ShapeDtypeStruct((B, S, D), jnp.float32),
        mesh=mesh,
        scratch_types=[
            pltpu.VMEM((SEQ_PER_W, S), jnp.int32),
            pltpu.VMEM((G, S, D), jnp.float32),
            pltpu.SemaphoreType.DMA((G,)),
            pltpu.SemaphoreType.DMA,
        ],
        compiler_params=pltpu.CompilerParams(use_tc_tiling_on_sc=False),
    )
    def emb(tid_hbm, table_hbm, out_hbm, idx_v, rows_v, gsem, wsem):
        wid = lax.axis_index("s") * 2 + lax.axis_index("c")
        seq0 = wid * SEQ_PER_W
        pltpu.sync_copy(tid_hbm.at[pl.ds(seq0, SEQ_PER_W)], idx_v)

        def body(grp, _):
            r0 = grp * G
            gathers = [
                pltpu.async_copy(
                    table_hbm.at[idx_v.at[r0 + b]], rows_v.at[b], gsem.at[b]
                )
                for b in range(G)
            ]
            wbs = []
            for b in range(G):
                gathers[b].wait()
                wbs.append(
                    pltpu.async_copy(
                        rows_v.at[b], out_hbm.at[seq0 + r0 + b], wsem
                    )
                )
            for wb in wbs:
                wb.wait()
            return ()

        lax.fori_loop(0, NGRP, body, ())

    return emb


_emb = _make_kernel()


@jax.jit
def kernel(token_ids, weight):
    return _emb(token_ids, weight)
